# baseline (speedup 1.0000x reference)
import functools

import jax
import jax.numpy as jnp
from jax.experimental import pallas as pl
from jax.experimental.pallas import tpu as pltpu

N = 10000
E = 160000
L = 512
BLK = 1000


def _layer1_body(x_ref, aggx_ref, deg_ref, w1l_ref, b1_ref, w1r_ref, h1_ref):
    deg = jnp.maximum(deg_ref[...], 1.0)
    a = aggx_ref[...] / deg
    h = a * w1l_ref[...] + b1_ref[...][None, :] + x_ref[...] * w1r_ref[...]
    h1_ref[...] = jnp.maximum(h, 0.0)


def _dense_body(agg2_ref, h1_ref, deg_ref, wcl_ref, bc_ref, wcr_ref,
                w1_ref, bl1_ref, w2_ref, bl2_ref, w3_ref, bl3_ref,
                wf_ref, bf_ref, out_ref):
    deg = jnp.maximum(deg_ref[...], 1.0)
    agg2 = agg2_ref[...] / deg
    t = jnp.dot(agg2, wcl_ref[...], preferred_element_type=jnp.float32)
    t += jnp.dot(h1_ref[...], wcr_ref[...], preferred_element_type=jnp.float32)
    t = jnp.maximum(t + bc_ref[...][None, :], 0.0)
    t = jnp.maximum(jnp.dot(t, w1_ref[...], preferred_element_type=jnp.float32)
                    + bl1_ref[...][None, :], 0.0)
    t = jnp.maximum(jnp.dot(t, w2_ref[...], preferred_element_type=jnp.float32)
                    + bl2_ref[...][None, :], 0.0)
    t = jnp.maximum(jnp.dot(t, w3_ref[...], preferred_element_type=jnp.float32)
                    + bl3_ref[...][None, :], 0.0)
    logits = jnp.dot(t, wf_ref[...], preferred_element_type=jnp.float32) \
        + bf_ref[...][None, :]
    m = jnp.max(logits, axis=1, keepdims=True)
    e = jnp.exp(logits - m)
    out_ref[...] = e / jnp.sum(e, axis=1, keepdims=True)


def _row_spec(width):
    return pl.BlockSpec((BLK, width), lambda i: (i, 0))


def _full_spec(shape):
    nd = len(shape)
    return pl.BlockSpec(shape, lambda i: (0,) * nd)


def kernel(x, edge_index, batch, W1l, b1, W1r, Wcl, bc, Wcr,
           Wlin1, blin1, Wlin2, blin2, Wlin3, blin3, Wfin, bfin):
    src = edge_index[0].astype(jnp.int32)
    dst = edge_index[1].astype(jnp.int32)
    xf = x[:, 0]

    deg = jax.ops.segment_sum(jnp.ones((E,), jnp.float32), dst, num_segments=N)
    aggx = jax.ops.segment_sum(xf[src], dst, num_segments=N)
    deg2 = deg[:, None]

    h1 = pl.pallas_call(
        _layer1_body,
        grid=(N // BLK,),
        in_specs=[_row_spec(1), _row_spec(1), _row_spec(1),
                  _full_spec((1, L)), _full_spec((L,)), _full_spec((1, L))],
        out_specs=_row_spec(L),
        out_shape=jax.ShapeDtypeStruct((N, L), jnp.float32),
    )(x, aggx[:, None], deg2, W1l, b1, W1r)

    agg2s = jax.ops.segment_sum(h1[src], dst, num_segments=N)

    out = pl.pallas_call(
        _dense_body,
        grid=(N // BLK,),
        in_specs=[_row_spec(L), _row_spec(L), _row_spec(1),
                  _full_spec((L, L)), _full_spec((L,)), _full_spec((L, L)),
                  _full_spec((L, 256)), _full_spec((256,)),
                  _full_spec((256, 128)), _full_spec((128,)),
                  _full_spec((128, 64)), _full_spec((64,)),
                  _full_spec((64, 2)), _full_spec((2,))],
        out_specs=_row_spec(2),
        out_shape=jax.ShapeDtypeStruct((N, 2), jnp.float32),
    )(agg2s, h1, deg2, Wcl, bc, Wcr, Wlin1, blin1, Wlin2, blin2,
      Wlin3, blin3, Wfin, bfin)
    return out


# speedup vs baseline: 3.8819x; 3.8819x over previous
import functools

import jax
import jax.numpy as jnp
from jax import lax
from jax.experimental import pallas as pl
from jax.experimental.pallas import tpu as pltpu
from jax.experimental.pallas import tpu_sc as plsc

N = 10000
E = 160000
L = 512
NC = 2
NS = 16
N_P = 10240
NCHUNK = 8
CW = 64
EG = N_P
NB = EG // 128
EW = EG // NC
BLK = 1024
NBLK = N_P // BLK

_mesh = plsc.VectorSubcoreMesh(core_axis_name="c", subcore_axis_name="s",
                               num_cores=NC, num_subcores=NS)


def _sc1_body(x_hbm, src_hbm, dst_hbm, aggx_hbm, deg_hbm,
              x_v, src_v, dst_v, acc_a, acc_d):
    c = lax.axis_index("c")
    s = lax.axis_index("s")
    w = s * NC + c
    pltpu.sync_copy(x_hbm, x_v)
    pltpu.sync_copy(src_hbm.at[s].at[pl.ds(c * EW, EW)], src_v)
    pltpu.sync_copy(dst_hbm.at[s].at[pl.ds(c * EW, EW)], dst_v)

    def zbody(i, _):
        z = jnp.zeros((16,), jnp.float32)
        acc_a[pl.ds(i * 16, 16)] = z
        acc_d[pl.ds(i * 16, 16)] = z
        return 0
    lax.fori_loop(0, N_P // 16, zbody, 0)

    ones = jnp.ones((16,), jnp.float32)

    def body(i, _):
        si = src_v[pl.ds(i * 16, 16)]
        di = dst_v[pl.ds(i * 16, 16)]
        vals = plsc.load_gather(x_v, [si])
        plsc.addupdate_scatter(acc_a, [di], vals)
        plsc.addupdate_scatter(acc_d, [di], ones)
        return 0
    lax.fori_loop(0, EW // 16, body, 0)

    pltpu.sync_copy(acc_a, aggx_hbm.at[w])
    pltpu.sync_copy(acc_d, deg_hbm.at[w])


_sc1 = functools.partial(
    pl.kernel,
    out_type=[jax.ShapeDtypeStruct((NC * NS, N_P), jnp.float32),
              jax.ShapeDtypeStruct((NC * NS, N_P), jnp.float32)],
    mesh=_mesh,
    scratch_types=[
        pltpu.VMEM((N_P,), jnp.float32),
        pltpu.VMEM((EW,), jnp.int32),
        pltpu.VMEM((EW,), jnp.int32),
        pltpu.VMEM((N_P,), jnp.float32),
        pltpu.VMEM((N_P,), jnp.float32),
    ],
    compiler_params=pltpu.CompilerParams(needs_layout_passes=False),
)(_sc1_body)


def _sc2_body(h1c_hbm, src_hbm, dst3_hbm, zeros_hbm, out_hbm,
              src_v, dst_v2, buf_a, buf_b, acc, sem_a, sem_b):
    c = lax.axis_index("c")
    s = lax.axis_index("s")
    rt = N_P // NS
    pltpu.sync_copy(src_hbm.at[s], src_v)
    pltpu.sync_copy(dst3_hbm.at[s], dst_v2)

    def add_off(off):
        def offbody(i, _):
            src_v[pl.ds(i * 16, 16)] = src_v[pl.ds(i * 16, 16)] + off
            return 0
        lax.fori_loop(0, EG // 16, offbody, 0)

    def mk(i, buf, sem):
        return pltpu.make_async_copy(
            h1c_hbm.at[src_v.at[pl.ds(i * 128, 128)]], buf, sem)

    def chunk_pass(chunk):
        pltpu.sync_copy(zeros_hbm, acc.at[pl.ds(s * rt, rt)])
        plsc.subcore_barrier()
        mk(0, buf_a, sem_a).start()
        mk(1, buf_b, sem_b).start()

        def pair(j, _):
            mk(2 * j, buf_a, sem_a).wait()
            pltpu.sync_copy(buf_a, acc.at[dst_v2.at[2 * j]], add=True)
            mk(2 * j + 2, buf_a, sem_a).start()
            mk(2 * j + 1, buf_b, sem_b).wait()
            pltpu.sync_copy(buf_b, acc.at[dst_v2.at[2 * j + 1]], add=True)
            mk(2 * j + 3, buf_b, sem_b).start()
            return 0
        lax.fori_loop(0, NB // 2 - 1, pair, 0)
        mk(NB - 2, buf_a, sem_a).wait()
        pltpu.sync_copy(buf_a, acc.at[dst_v2.at[NB - 2]], add=True)
        mk(NB - 1, buf_b, sem_b).wait()
        pltpu.sync_copy(buf_b, acc.at[dst_v2.at[NB - 1]], add=True)
        plsc.subcore_barrier()
        pltpu.sync_copy(acc.at[pl.ds(s * rt, rt)],
                        out_hbm.at[pl.ds(chunk * N_P + s * rt, rt)])
        plsc.subcore_barrier()

    add_off((NCHUNK // NC) * c * N_P)
    chunk_pass((NCHUNK // NC) * c)
    for j in range(1, NCHUNK // NC):
        add_off(N_P)
        chunk_pass((NCHUNK // NC) * c + j)


_sc2 = functools.partial(
    pl.kernel,
    out_type=jax.ShapeDtypeStruct((NCHUNK * N_P, CW), jnp.float32),
    mesh=_mesh,
    scratch_types=[
        pltpu.VMEM((EG,), jnp.int32),
        pltpu.VMEM((NB, 128), jnp.int32),
        pltpu.VMEM((128, CW), jnp.float32),
        pltpu.VMEM((128, CW), jnp.float32),
        pltpu.VMEM_SHARED((N_P, CW), jnp.float32),
        pltpu.SemaphoreType.DMA,
        pltpu.SemaphoreType.DMA,
    ],
    compiler_params=pltpu.CompilerParams(needs_layout_passes=False,
                                         use_tc_tiling_on_sc=False),
)(_sc2_body)


def _layer1_body(x_ref, aggx_ref, degp_ref, w1l_ref, b1_ref, w1r_ref,
                 h1c_ref, deg_ref):
    aggx = jnp.sum(aggx_ref[...], axis=0)
    deg = jnp.maximum(jnp.sum(degp_ref[...], axis=0), 1.0)
    a = aggx / deg
    h = (a[:, None] * w1l_ref[0] + b1_ref[0]
         + x_ref[...] * w1r_ref[0])
    h1c_ref[...] = jnp.maximum(h, 0.0)
    deg_ref[...] = deg[:, None]


def _dense_body(*refs):
    a_refs = refs[0:NCHUNK]
    h_refs = refs[NCHUNK:2 * NCHUNK]
    (deg_ref, wcl_ref, bc_ref, wcr_ref, w1_ref, bl1_ref, w2_ref, bl2_ref,
     w3_ref, bl3_ref, wf_ref, bf_ref, out_ref) = refs[2 * NCHUNK:]
    deg = jnp.maximum(deg_ref[...], 1.0)
    agg2 = jnp.concatenate([r[...] for r in a_refs], axis=1) / deg
    h1 = jnp.concatenate([r[...] for r in h_refs], axis=1)
    t = jnp.dot(agg2, wcl_ref[...], preferred_element_type=jnp.float32)
    t += jnp.dot(h1, wcr_ref[...], preferred_element_type=jnp.float32)
    t = jnp.maximum(t + bc_ref[...][None, :], 0.0)
    t = jnp.maximum(jnp.dot(t, w1_ref[...], preferred_element_type=jnp.float32)
                    + bl1_ref[...][None, :], 0.0)
    t = jnp.maximum(jnp.dot(t, w2_ref[...], preferred_element_type=jnp.float32)
                    + bl2_ref[...][None, :], 0.0)
    t = jnp.maximum(jnp.dot(t, w3_ref[...], preferred_element_type=jnp.float32)
                    + bl3_ref[...][None, :], 0.0)
    logits = jnp.dot(t, wf_ref[...], preferred_element_type=jnp.float32) \
        + bf_ref[...][None, :]
    m = jnp.max(logits, axis=1, keepdims=True)
    e = jnp.exp(logits - m)
    out_ref[...] = e / jnp.sum(e, axis=1, keepdims=True)


def kernel(x, edge_index, batch, W1l, b1, W1r, Wcl, bc, Wcr,
           Wlin1, blin1, Wlin2, blin2, Wlin3, blin3, Wfin, bfin):
    src = edge_index[0].astype(jnp.int32).reshape(NS, N)
    dst = edge_index[1].astype(jnp.int32).reshape(NS, N)
    src2 = jnp.pad(src, ((0, 0), (0, EG - N)))
    dst2 = jnp.pad(dst, ((0, 0), (0, EG - N)), constant_values=N)
    dst3 = dst2.reshape(NS, NB, 128)
    x_p = jnp.pad(x[:, 0], (0, N_P - N))
    zeros_blk = jnp.zeros((N_P // NS, CW), jnp.float32)

    aggx_p, deg_p = _sc1(x_p, src2, dst2)

    h1c, deg = pl.pallas_call(
        _layer1_body,
        grid=(NBLK, NCHUNK),
        in_specs=[
            pl.BlockSpec((BLK, 1), lambda i, c: (i, 0)),
            pl.BlockSpec((NC * NS, BLK), lambda i, c: (0, i)),
            pl.BlockSpec((NC * NS, BLK), lambda i, c: (0, i)),
            pl.BlockSpec((1, 1, CW), lambda i, c: (c, 0, 0)),
            pl.BlockSpec((1, 1, CW), lambda i, c: (c, 0, 0)),
            pl.BlockSpec((1, 1, CW), lambda i, c: (c, 0, 0)),
        ],
        out_specs=[
            pl.BlockSpec((BLK, CW), lambda i, c: (c * NBLK + i, 0)),
            pl.BlockSpec((BLK, 1), lambda i, c: (i, 0)),
        ],
        out_shape=[
            jax.ShapeDtypeStruct((NCHUNK * N_P, CW), jnp.float32),
            jax.ShapeDtypeStruct((N_P, 1), jnp.float32),
        ],
    )(jnp.pad(x, ((0, N_P - N), (0, 0))), aggx_p, deg_p,
      W1l.reshape(NCHUNK, 1, CW), b1.reshape(NCHUNK, 1, CW),
      W1r.reshape(NCHUNK, 1, CW))

    agg2 = _sc2(h1c, src2, dst3, zeros_blk)

    def _rows(c):
        return pl.BlockSpec((BLK, CW), lambda i, c=c: (c * NBLK + i, 0))

    out = pl.pallas_call(
        _dense_body,
        grid=(NBLK,),
        in_specs=(
            [_rows(c) for c in range(NCHUNK)]
            + [_rows(c) for c in range(NCHUNK)]
            + [
                pl.BlockSpec((BLK, 1), lambda i: (i, 0)),
                pl.BlockSpec((L, L), lambda i: (0, 0)),
                pl.BlockSpec((L,), lambda i: (0,)),
                pl.BlockSpec((L, L), lambda i: (0, 0)),
                pl.BlockSpec((L, 256), lambda i: (0, 0)),
                pl.BlockSpec((256,), lambda i: (0,)),
                pl.BlockSpec((256, 128), lambda i: (0, 0)),
                pl.BlockSpec((128,), lambda i: (0,)),
                pl.BlockSpec((128, 64), lambda i: (0, 0)),
                pl.BlockSpec((64,), lambda i: (0,)),
                pl.BlockSpec((64, 2), lambda i: (0, 0)),
                pl.BlockSpec((2,), lambda i: (0,)),
            ]
        ),
        out_specs=pl.BlockSpec((BLK, 2), lambda i: (i, 0)),
        out_shape=jax.ShapeDtypeStruct((N_P, 2), jnp.float32),
    )(*([agg2] * NCHUNK), *([h1c] * NCHUNK), deg,
      Wcl, bc, Wcr, Wlin1, blin1, Wlin2, blin2, Wlin3, blin3, Wfin, bfin)
    return out[:N]


# speedup vs baseline: 4.1882x; 1.0789x over previous
import functools

import jax
import jax.numpy as jnp
from jax import lax
from jax.experimental import pallas as pl
from jax.experimental.pallas import tpu as pltpu
from jax.experimental.pallas import tpu_sc as plsc

N = 10000
E = 160000
L = 512
NC = 2
NS = 16
N_P = 10240
NCHUNK = 8
CW = 64
EG = N_P
NB = EG // 128
EW = EG // NC
BLK = 1024
NBLK = N_P // BLK

_mesh = plsc.VectorSubcoreMesh(core_axis_name="c", subcore_axis_name="s",
                               num_cores=NC, num_subcores=NS)


def _sc1_body(x_hbm, src_hbm, dst_hbm, aggx_hbm, deg_hbm,
              x_v, src_v, dst_v, acc_a, acc_d):
    c = lax.axis_index("c")
    s = lax.axis_index("s")
    w = s * NC + c
    pltpu.sync_copy(x_hbm, x_v)
    pltpu.sync_copy(src_hbm.at[s].at[pl.ds(c * EW, EW)], src_v)
    pltpu.sync_copy(dst_hbm.at[s].at[pl.ds(c * EW, EW)], dst_v)

    def zbody(i, _):
        z = jnp.zeros((16,), jnp.float32)
        acc_a[pl.ds(i * 16, 16)] = z
        acc_d[pl.ds(i * 16, 16)] = z
        return 0
    lax.fori_loop(0, N_P // 16, zbody, 0)

    ones = jnp.ones((16,), jnp.float32)

    def body(i, _):
        si = src_v[pl.ds(i * 16, 16)]
        di = dst_v[pl.ds(i * 16, 16)]
        vals = plsc.load_gather(x_v, [si])
        plsc.addupdate_scatter(acc_a, [di], vals)
        plsc.addupdate_scatter(acc_d, [di], ones)
        return 0
    lax.fori_loop(0, EW // 16, body, 0)

    pltpu.sync_copy(acc_a, aggx_hbm.at[w])
    pltpu.sync_copy(acc_d, deg_hbm.at[w])


_sc1 = functools.partial(
    pl.kernel,
    out_type=[jax.ShapeDtypeStruct((NC * NS, N_P), jnp.float32),
              jax.ShapeDtypeStruct((NC * NS, N_P), jnp.float32)],
    mesh=_mesh,
    scratch_types=[
        pltpu.VMEM((N_P,), jnp.float32),
        pltpu.VMEM((EW,), jnp.int32),
        pltpu.VMEM((EW,), jnp.int32),
        pltpu.VMEM((N_P,), jnp.float32),
        pltpu.VMEM((N_P,), jnp.float32),
    ],
    compiler_params=pltpu.CompilerParams(needs_layout_passes=False),
)(_sc1_body)


NBUF = 4


def _sc2_body(h1c_hbm, src_hbm, dst3_hbm, out_hbm,
              src_v, dst_v2, zbuf, b0, b1, b2, b3, acc,
              zsem, g0, g1, g2, g3, s0, s1, s2, s3):
    c = lax.axis_index("c")
    s = lax.axis_index("s")
    bufs = (b0, b1, b2, b3)
    gsems = (g0, g1, g2, g3)
    ssems = (s0, s1, s2, s3)
    rt = N_P // NS
    pltpu.sync_copy(src_hbm.at[s], src_v)
    pltpu.sync_copy(dst3_hbm.at[s], dst_v2)

    def zb(i, _):
        zbuf[i // 4, pl.ds((i % 4) * 16, 16)] = jnp.zeros((16,), jnp.float32)
        return 0
    lax.fori_loop(0, 128 * CW // 16, zb, 0)

    def add_off(off):
        def offbody(i, _):
            src_v[pl.ds(i * 16, 16)] = src_v[pl.ds(i * 16, 16)] + off
            return 0
        lax.fori_loop(0, EG // 16, offbody, 0)

    def mk_g(i, buf, sem):
        return pltpu.make_async_copy(
            h1c_hbm.at[src_v.at[pl.ds(i * 128, 128)]], buf, sem)

    def chunk_pass(chunk):
        zdescs = [pltpu.make_async_copy(
            zbuf, acc.at[pl.ds(s * rt + k * 128, 128)], zsem)
            for k in range(rt // 128)]
        for d in zdescs:
            d.start()
        for d in zdescs:
            d.wait()
        plsc.subcore_barrier()
        for b in range(NBUF):
            mk_g(b, bufs[b], gsems[b]).start()

        def rnd(r, _):
            sds = []
            for b in range(NBUF):
                j = NBUF * r + b
                mk_g(j, bufs[b], gsems[b]).wait()
                sds.append(pltpu.async_copy(
                    bufs[b], acc.at[dst_v2.at[j]], ssems[b], add=True))
            for b in range(NBUF):
                sds[b].wait()
                mk_g(NBUF * r + b + NBUF, bufs[b], gsems[b]).start()
            return 0
        lax.fori_loop(0, NB // NBUF - 1, rnd, 0)
        tds = []
        for b in range(NBUF):
            j = NB - NBUF + b
            mk_g(j, bufs[b], gsems[b]).wait()
            tds.append(pltpu.async_copy(
                bufs[b], acc.at[dst_v2.at[j]], ssems[b], add=True))
        for d in tds:
            d.wait()
        plsc.subcore_barrier()
        pltpu.sync_copy(acc.at[pl.ds(s * rt, rt)],
                        out_hbm.at[pl.ds(chunk * N_P + s * rt, rt)])
        plsc.subcore_barrier()

    add_off((NCHUNK // NC) * c * N_P)
    chunk_pass((NCHUNK // NC) * c)
    for j in range(1, NCHUNK // NC):
        add_off(N_P)
        chunk_pass((NCHUNK // NC) * c + j)


_sc2 = functools.partial(
    pl.kernel,
    out_type=jax.ShapeDtypeStruct((NCHUNK * N_P, CW), jnp.float32),
    mesh=_mesh,
    scratch_types=(
        [pltpu.VMEM((EG,), jnp.int32),
         pltpu.VMEM((NB, 128), jnp.int32),
         pltpu.VMEM((128, CW), jnp.float32)]
        + [pltpu.VMEM((128, CW), jnp.float32) for _ in range(NBUF)]
        + [pltpu.VMEM_SHARED((N_P, CW), jnp.float32)]
        + [pltpu.SemaphoreType.DMA for _ in range(2 * NBUF + 1)]
    ),
    compiler_params=pltpu.CompilerParams(needs_layout_passes=False,
                                         use_tc_tiling_on_sc=False),
)(_sc2_body)


def _layer1_body(x_ref, aggx_ref, degp_ref, w1l_ref, b1_ref, w1r_ref,
                 h1c_ref, deg_ref):
    aggx = jnp.sum(aggx_ref[...], axis=0)
    deg = jnp.maximum(jnp.sum(degp_ref[...], axis=0), 1.0)
    a = aggx / deg
    h = (a[:, None] * w1l_ref[0] + b1_ref[0]
         + x_ref[...] * w1r_ref[0])
    h1c_ref[...] = jnp.maximum(h, 0.0)
    deg_ref[...] = deg[:, None]


def _dense_body(*refs):
    a_refs = refs[0:NCHUNK]
    h_refs = refs[NCHUNK:2 * NCHUNK]
    (deg_ref, wcl_ref, bc_ref, wcr_ref, w1_ref, bl1_ref, w2_ref, bl2_ref,
     w3_ref, bl3_ref, wf_ref, bf_ref, out_ref) = refs[2 * NCHUNK:]
    deg = jnp.maximum(deg_ref[...], 1.0)
    agg2 = jnp.concatenate([r[...] for r in a_refs], axis=1) / deg
    h1 = jnp.concatenate([r[...] for r in h_refs], axis=1)
    t = jnp.dot(agg2, wcl_ref[...], preferred_element_type=jnp.float32)
    t += jnp.dot(h1, wcr_ref[...], preferred_element_type=jnp.float32)
    t = jnp.maximum(t + bc_ref[...][None, :], 0.0)
    t = jnp.maximum(jnp.dot(t, w1_ref[...], preferred_element_type=jnp.float32)
                    + bl1_ref[...][None, :], 0.0)
    t = jnp.maximum(jnp.dot(t, w2_ref[...], preferred_element_type=jnp.float32)
                    + bl2_ref[...][None, :], 0.0)
    t = jnp.maximum(jnp.dot(t, w3_ref[...], preferred_element_type=jnp.float32)
                    + bl3_ref[...][None, :], 0.0)
    logits = jnp.dot(t, wf_ref[...], preferred_element_type=jnp.float32) \
        + bf_ref[...][None, :]
    m = jnp.max(logits, axis=1, keepdims=True)
    e = jnp.exp(logits - m)
    out_ref[...] = e / jnp.sum(e, axis=1, keepdims=True)


def kernel(x, edge_index, batch, W1l, b1, W1r, Wcl, bc, Wcr,
           Wlin1, blin1, Wlin2, blin2, Wlin3, blin3, Wfin, bfin):
    src = edge_index[0].astype(jnp.int32).reshape(NS, N)
    dst = edge_index[1].astype(jnp.int32).reshape(NS, N)
    src2 = jnp.pad(src, ((0, 0), (0, EG - N)))
    dst2 = jnp.pad(dst, ((0, 0), (0, EG - N)), constant_values=N)
    dst3 = dst2.reshape(NS, NB, 128)
    x_p = jnp.pad(x[:, 0], (0, N_P - N))

    aggx_p, deg_p = _sc1(x_p, src2, dst2)

    h1c, deg = pl.pallas_call(
        _layer1_body,
        grid=(NBLK, NCHUNK),
        in_specs=[
            pl.BlockSpec((BLK, 1), lambda i, c: (i, 0)),
            pl.BlockSpec((NC * NS, BLK), lambda i, c: (0, i)),
            pl.BlockSpec((NC * NS, BLK), lambda i, c: (0, i)),
            pl.BlockSpec((1, 1, CW), lambda i, c: (c, 0, 0)),
            pl.BlockSpec((1, 1, CW), lambda i, c: (c, 0, 0)),
            pl.BlockSpec((1, 1, CW), lambda i, c: (c, 0, 0)),
        ],
        out_specs=[
            pl.BlockSpec((BLK, CW), lambda i, c: (c * NBLK + i, 0)),
            pl.BlockSpec((BLK, 1), lambda i, c: (i, 0)),
        ],
        out_shape=[
            jax.ShapeDtypeStruct((NCHUNK * N_P, CW), jnp.float32),
            jax.ShapeDtypeStruct((N_P, 1), jnp.float32),
        ],
    )(jnp.pad(x, ((0, N_P - N), (0, 0))), aggx_p, deg_p,
      W1l.reshape(NCHUNK, 1, CW), b1.reshape(NCHUNK, 1, CW),
      W1r.reshape(NCHUNK, 1, CW))

    agg2 = _sc2(h1c, src2, dst3)

    def _rows(c):
        return pl.BlockSpec((BLK, CW), lambda i, c=c: (c * NBLK + i, 0))

    out = pl.pallas_call(
        _dense_body,
        grid=(NBLK,),
        in_specs=(
            [_rows(c) for c in range(NCHUNK)]
            + [_rows(c) for c in range(NCHUNK)]
            + [
                pl.BlockSpec((BLK, 1), lambda i: (i, 0)),
                pl.BlockSpec((L, L), lambda i: (0, 0)),
                pl.BlockSpec((L,), lambda i: (0,)),
                pl.BlockSpec((L, L), lambda i: (0, 0)),
                pl.BlockSpec((L, 256), lambda i: (0, 0)),
                pl.BlockSpec((256,), lambda i: (0,)),
                pl.BlockSpec((256, 128), lambda i: (0, 0)),
                pl.BlockSpec((128,), lambda i: (0,)),
                pl.BlockSpec((128, 64), lambda i: (0, 0)),
                pl.BlockSpec((64,), lambda i: (0,)),
                pl.BlockSpec((64, 2), lambda i: (0, 0)),
                pl.BlockSpec((2,), lambda i: (0,)),
            ]
        ),
        out_specs=pl.BlockSpec((BLK, 2), lambda i: (i, 0)),
        out_shape=jax.ShapeDtypeStruct((N_P, 2), jnp.float32),
    )(*([agg2] * NCHUNK), *([h1c] * NCHUNK), deg,
      Wcl, bc, Wcr, Wlin1, blin1, Wlin2, blin2, Wlin3, blin3, Wfin, bfin)
    return out[:N]


# speedup vs baseline: 4.2595x; 1.0170x over previous
import functools

import jax
import jax.numpy as jnp
from jax import lax
from jax.experimental import pallas as pl
from jax.experimental.pallas import tpu as pltpu
from jax.experimental.pallas import tpu_sc as plsc

N = 10000
E = 160000
L = 512
NC = 2
NS = 16
N_P = 10240
NCHUNK = 8
CW = 64
EG = N_P
NB = EG // 128
EW = EG // NC
BLK = 1024
NBLK = N_P // BLK

_mesh = plsc.VectorSubcoreMesh(core_axis_name="c", subcore_axis_name="s",
                               num_cores=NC, num_subcores=NS)


def _sc1_body(x_hbm, src_hbm, dst_hbm, aggx_hbm, deg_hbm,
              x_v, src_v, dst_v, acc_a, acc_d):
    c = lax.axis_index("c")
    s = lax.axis_index("s")
    w = s * NC + c
    pltpu.sync_copy(x_hbm, x_v)
    pltpu.sync_copy(src_hbm.at[s].at[pl.ds(c * EW, EW)], src_v)
    pltpu.sync_copy(dst_hbm.at[s].at[pl.ds(c * EW, EW)], dst_v)

    def zbody(i, _):
        z = jnp.zeros((16,), jnp.float32)
        acc_a[pl.ds(i * 16, 16)] = z
        acc_d[pl.ds(i * 16, 16)] = z
        return 0
    lax.fori_loop(0, N_P // 16, zbody, 0)

    ones = jnp.ones((16,), jnp.float32)

    def body(i, _):
        si = src_v[pl.ds(i * 16, 16)]
        di = dst_v[pl.ds(i * 16, 16)]
        vals = plsc.load_gather(x_v, [si])
        plsc.addupdate_scatter(acc_a, [di], vals)
        plsc.addupdate_scatter(acc_d, [di], ones)
        return 0
    lax.fori_loop(0, EW // 16, body, 0)

    pltpu.sync_copy(acc_a, aggx_hbm.at[w])
    pltpu.sync_copy(acc_d, deg_hbm.at[w])


_sc1 = functools.partial(
    pl.kernel,
    out_type=[jax.ShapeDtypeStruct((NC * NS, N_P), jnp.float32),
              jax.ShapeDtypeStruct((NC * NS, N_P), jnp.float32)],
    mesh=_mesh,
    scratch_types=[
        pltpu.VMEM((N_P,), jnp.float32),
        pltpu.VMEM((EW,), jnp.int32),
        pltpu.VMEM((EW,), jnp.int32),
        pltpu.VMEM((N_P,), jnp.float32),
        pltpu.VMEM((N_P,), jnp.float32),
    ],
    compiler_params=pltpu.CompilerParams(needs_layout_passes=False),
)(_sc1_body)


NBUF = 8


def _sc2_body(h1c_hbm, src_hbm, dst3_hbm, out_hbm,
              src_v, dst_v2, b0, b1, b2, b3, b4, b5, b6, b7, acc,
              zsem, g0, g1, g2, g3, g4, g5, g6, g7,
              s0, s1, s2, s3, s4, s5, s6, s7):
    c = lax.axis_index("c")
    s = lax.axis_index("s")
    zbuf = b0
    bufs = (b0, b1, b2, b3, b4, b5, b6, b7)
    gsems = (g0, g1, g2, g3, g4, g5, g6, g7)
    ssems = (s0, s1, s2, s3, s4, s5, s6, s7)
    rt = N_P // NS
    pltpu.sync_copy(src_hbm.at[s], src_v)
    pltpu.sync_copy(dst3_hbm.at[s], dst_v2)

    def add_off(off):
        def offbody(i, _):
            src_v[pl.ds(i * 16, 16)] = src_v[pl.ds(i * 16, 16)] + off
            return 0
        lax.fori_loop(0, EG // 16, offbody, 0)

    def mk_g(i, buf, sem):
        return pltpu.make_async_copy(
            h1c_hbm.at[src_v.at[pl.ds(i * 128, 128)]], buf, sem)

    def chunk_pass(chunk):
        def zb(i, _):
            zbuf[i // (CW // 16), pl.ds((i % (CW // 16)) * 16, 16)] = (
                jnp.zeros((16,), jnp.float32))
            return 0
        lax.fori_loop(0, 128 * CW // 16, zb, 0)
        zdescs = [pltpu.make_async_copy(
            zbuf, acc.at[pl.ds(s * rt + k * 128, 128)], zsem)
            for k in range(rt // 128)]
        for d in zdescs:
            d.start()
        for d in zdescs:
            d.wait()
        plsc.subcore_barrier()
        for b in range(NBUF):
            mk_g(b, bufs[b], gsems[b]).start()

        def rnd(r, _):
            sds = []
            for b in range(NBUF):
                j = NBUF * r + b
                mk_g(j, bufs[b], gsems[b]).wait()
                sds.append(pltpu.async_copy(
                    bufs[b], acc.at[dst_v2.at[j]], ssems[b], add=True))
            for b in range(NBUF):
                sds[b].wait()
                mk_g(NBUF * r + b + NBUF, bufs[b], gsems[b]).start()
            return 0
        lax.fori_loop(0, NB // NBUF - 1, rnd, 0)
        tds = []
        for b in range(NBUF):
            j = NB - NBUF + b
            mk_g(j, bufs[b], gsems[b]).wait()
            tds.append(pltpu.async_copy(
                bufs[b], acc.at[dst_v2.at[j]], ssems[b], add=True))
        for d in tds:
            d.wait()
        plsc.subcore_barrier()
        pltpu.sync_copy(acc.at[pl.ds(s * rt, rt)],
                        out_hbm.at[pl.ds(chunk * N_P + s * rt, rt)])
        plsc.subcore_barrier()

    add_off((NCHUNK // NC) * c * N_P)
    chunk_pass((NCHUNK // NC) * c)
    for j in range(1, NCHUNK // NC):
        add_off(N_P)
        chunk_pass((NCHUNK // NC) * c + j)


_sc2 = functools.partial(
    pl.kernel,
    out_type=jax.ShapeDtypeStruct((NCHUNK * N_P, CW), jnp.float32),
    mesh=_mesh,
    scratch_types=(
        [pltpu.VMEM((EG,), jnp.int32),
         pltpu.VMEM((NB, 128), jnp.int32)]
        + [pltpu.VMEM((128, CW), jnp.float32) for _ in range(NBUF)]
        + [pltpu.VMEM_SHARED((N_P, CW), jnp.float32)]
        + [pltpu.SemaphoreType.DMA for _ in range(2 * NBUF + 1)]
    ),
    compiler_params=pltpu.CompilerParams(needs_layout_passes=False,
                                         use_tc_tiling_on_sc=False),
)(_sc2_body)


def _layer1_body(x_ref, aggx_ref, degp_ref, w1l_ref, b1_ref, w1r_ref,
                 h1c_ref, deg_ref):
    aggx = jnp.sum(aggx_ref[...], axis=0)
    deg = jnp.maximum(jnp.sum(degp_ref[...], axis=0), 1.0)
    a = aggx / deg
    h = (a[:, None] * w1l_ref[0] + b1_ref[0]
         + x_ref[...] * w1r_ref[0])
    h1c_ref[...] = jnp.maximum(h, 0.0)
    deg_ref[...] = deg[:, None]


def _dense_body(*refs):
    a_refs = refs[0:NCHUNK]
    h_refs = refs[NCHUNK:2 * NCHUNK]
    (deg_ref, wcl_ref, bc_ref, wcr_ref, w1_ref, bl1_ref, w2_ref, bl2_ref,
     w3_ref, bl3_ref, wf_ref, bf_ref, out_ref) = refs[2 * NCHUNK:]
    deg = jnp.maximum(deg_ref[...], 1.0)
    agg2 = jnp.concatenate([r[...] for r in a_refs], axis=1) / deg
    h1 = jnp.concatenate([r[...] for r in h_refs], axis=1)
    t = jnp.dot(agg2, wcl_ref[...], preferred_element_type=jnp.float32)
    t += jnp.dot(h1, wcr_ref[...], preferred_element_type=jnp.float32)
    t = jnp.maximum(t + bc_ref[...][None, :], 0.0)
    t = jnp.maximum(jnp.dot(t, w1_ref[...], preferred_element_type=jnp.float32)
                    + bl1_ref[...][None, :], 0.0)
    t = jnp.maximum(jnp.dot(t, w2_ref[...], preferred_element_type=jnp.float32)
                    + bl2_ref[...][None, :], 0.0)
    t = jnp.maximum(jnp.dot(t, w3_ref[...], preferred_element_type=jnp.float32)
                    + bl3_ref[...][None, :], 0.0)
    logits = jnp.dot(t, wf_ref[...], preferred_element_type=jnp.float32) \
        + bf_ref[...][None, :]
    m = jnp.max(logits, axis=1, keepdims=True)
    e = jnp.exp(logits - m)
    out_ref[...] = e / jnp.sum(e, axis=1, keepdims=True)


def kernel(x, edge_index, batch, W1l, b1, W1r, Wcl, bc, Wcr,
           Wlin1, blin1, Wlin2, blin2, Wlin3, blin3, Wfin, bfin):
    src = edge_index[0].astype(jnp.int32).reshape(NS, N)
    dst = edge_index[1].astype(jnp.int32).reshape(NS, N)
    src2 = jnp.pad(src, ((0, 0), (0, EG - N)))
    dst2 = jnp.pad(dst, ((0, 0), (0, EG - N)), constant_values=N)
    dst3 = dst2.reshape(NS, NB, 128)
    x_p = jnp.pad(x[:, 0], (0, N_P - N))

    aggx_p, deg_p = _sc1(x_p, src2, dst2)

    h1c, deg = pl.pallas_call(
        _layer1_body,
        grid=(NBLK, NCHUNK),
        in_specs=[
            pl.BlockSpec((BLK, 1), lambda i, c: (i, 0)),
            pl.BlockSpec((NC * NS, BLK), lambda i, c: (0, i)),
            pl.BlockSpec((NC * NS, BLK), lambda i, c: (0, i)),
            pl.BlockSpec((1, 1, CW), lambda i, c: (c, 0, 0)),
            pl.BlockSpec((1, 1, CW), lambda i, c: (c, 0, 0)),
            pl.BlockSpec((1, 1, CW), lambda i, c: (c, 0, 0)),
        ],
        out_specs=[
            pl.BlockSpec((BLK, CW), lambda i, c: (c * NBLK + i, 0)),
            pl.BlockSpec((BLK, 1), lambda i, c: (i, 0)),
        ],
        out_shape=[
            jax.ShapeDtypeStruct((NCHUNK * N_P, CW), jnp.float32),
            jax.ShapeDtypeStruct((N_P, 1), jnp.float32),
        ],
    )(jnp.pad(x, ((0, N_P - N), (0, 0))), aggx_p, deg_p,
      W1l.reshape(NCHUNK, 1, CW), b1.reshape(NCHUNK, 1, CW),
      W1r.reshape(NCHUNK, 1, CW))

    agg2 = _sc2(h1c, src2, dst3)

    def _rows(c):
        return pl.BlockSpec((BLK, CW), lambda i, c=c: (c * NBLK + i, 0))

    out = pl.pallas_call(
        _dense_body,
        grid=(NBLK,),
        in_specs=(
            [_rows(c) for c in range(NCHUNK)]
            + [_rows(c) for c in range(NCHUNK)]
            + [
                pl.BlockSpec((BLK, 1), lambda i: (i, 0)),
                pl.BlockSpec((L, L), lambda i: (0, 0)),
                pl.BlockSpec((L,), lambda i: (0,)),
                pl.BlockSpec((L, L), lambda i: (0, 0)),
                pl.BlockSpec((L, 256), lambda i: (0, 0)),
                pl.BlockSpec((256,), lambda i: (0,)),
                pl.BlockSpec((256, 128), lambda i: (0, 0)),
                pl.BlockSpec((128,), lambda i: (0,)),
                pl.BlockSpec((128, 64), lambda i: (0, 0)),
                pl.BlockSpec((64,), lambda i: (0,)),
                pl.BlockSpec((64, 2), lambda i: (0, 0)),
                pl.BlockSpec((2,), lambda i: (0,)),
            ]
        ),
        out_specs=pl.BlockSpec((BLK, 2), lambda i: (i, 0)),
        out_shape=jax.ShapeDtypeStruct((N_P, 2), jnp.float32),
    )(*([agg2] * NCHUNK), *([h1c] * NCHUNK), deg,
      Wcl, bc, Wcr, Wlin1, blin1, Wlin2, blin2, Wlin3, blin3, Wfin, bfin)
    return out[:N]


# speedup vs baseline: 4.3719x; 1.0264x over previous
import functools

import jax
import jax.numpy as jnp
from jax import lax
from jax.experimental import pallas as pl
from jax.experimental.pallas import tpu as pltpu
from jax.experimental.pallas import tpu_sc as plsc

N = 10000
E = 160000
L = 512
NC = 2
NS = 16
N_P = 10240
NCHUNK = 4
CW = 128
EG = N_P
NB = EG // 128
EW = EG // NC
BLK = 1024
NBLK = N_P // BLK

_mesh = plsc.VectorSubcoreMesh(core_axis_name="c", subcore_axis_name="s",
                               num_cores=NC, num_subcores=NS)


def _sc1_body(x_hbm, src_hbm, dst_hbm, aggx_hbm, deg_hbm,
              x_v, src_v, dst_v, acc_a, acc_d):
    c = lax.axis_index("c")
    s = lax.axis_index("s")
    w = s * NC + c
    pltpu.sync_copy(x_hbm, x_v)
    pltpu.sync_copy(src_hbm.at[s].at[pl.ds(c * EW, EW)], src_v)
    pltpu.sync_copy(dst_hbm.at[s].at[pl.ds(c * EW, EW)], dst_v)

    def zbody(i, _):
        z = jnp.zeros((16,), jnp.float32)
        acc_a[pl.ds(i * 16, 16)] = z
        acc_d[pl.ds(i * 16, 16)] = z
        return 0
    lax.fori_loop(0, N_P // 16, zbody, 0)

    ones = jnp.ones((16,), jnp.float32)

    def body(i, _):
        si = src_v[pl.ds(i * 16, 16)]
        di = dst_v[pl.ds(i * 16, 16)]
        vals = plsc.load_gather(x_v, [si])
        plsc.addupdate_scatter(acc_a, [di], vals)
        plsc.addupdate_scatter(acc_d, [di], ones)
        return 0
    lax.fori_loop(0, EW // 16, body, 0)

    pltpu.sync_copy(acc_a, aggx_hbm.at[w])
    pltpu.sync_copy(acc_d, deg_hbm.at[w])


_sc1 = functools.partial(
    pl.kernel,
    out_type=[jax.ShapeDtypeStruct((NC * NS, N_P), jnp.float32),
              jax.ShapeDtypeStruct((NC * NS, N_P), jnp.float32)],
    mesh=_mesh,
    scratch_types=[
        pltpu.VMEM((N_P,), jnp.float32),
        pltpu.VMEM((EW,), jnp.int32),
        pltpu.VMEM((EW,), jnp.int32),
        pltpu.VMEM((N_P,), jnp.float32),
        pltpu.VMEM((N_P,), jnp.float32),
    ],
    compiler_params=pltpu.CompilerParams(needs_layout_passes=False),
)(_sc1_body)


NBUF = 2
NQ = 4
QB = NB // NQ


def _sc2_body(h1c_hbm, src_hbm, dst3_hbm, out_hbm,
              src_v, dst_q, b0, b1, acc,
              zsem, g0, g1, s0, s1):
    c = lax.axis_index("c")
    s = lax.axis_index("s")
    zbuf = b0
    bufs = (b0, b1)
    gsems = (g0, g1)
    ssems = (s0, s1)
    rt = N_P // NS
    pltpu.sync_copy(src_hbm.at[s], src_v)

    def add_off(off):
        def offbody(i, _):
            src_v[pl.ds(i * 16, 16)] = src_v[pl.ds(i * 16, 16)] + off
            return 0
        lax.fori_loop(0, EG // 16, offbody, 0)

    def mk_g(i, buf, sem):
        return pltpu.make_async_copy(
            h1c_hbm.at[src_v.at[pl.ds(i * 128, 128)]], buf, sem)

    def chunk_pass(chunk):
        def zb(i, _):
            zbuf[i // (CW // 16), pl.ds((i % (CW // 16)) * 16, 16)] = (
                jnp.zeros((16,), jnp.float32))
            return 0
        lax.fori_loop(0, 128 * CW // 16, zb, 0)
        zdescs = [pltpu.make_async_copy(
            zbuf, acc.at[pl.ds(s * rt + k * 128, 128)], zsem)
            for k in range(rt // 128)]
        for d in zdescs:
            d.start()
        for d in zdescs:
            d.wait()
        plsc.subcore_barrier()
        for q in range(NQ):
            base = q * QB
            pltpu.sync_copy(dst3_hbm.at[s].at[pl.ds(base, QB)], dst_q)
            for b in range(NBUF):
                mk_g(base + b, bufs[b], gsems[b]).start()

            def rnd(r, _):
                sds = []
                for b in range(NBUF):
                    lj = NBUF * r + b
                    mk_g(base + lj, bufs[b], gsems[b]).wait()
                    sds.append(pltpu.async_copy(
                        bufs[b], acc.at[dst_q.at[lj]], ssems[b], add=True))
                for b in range(NBUF):
                    sds[b].wait()
                    mk_g(base + NBUF * r + b + NBUF, bufs[b],
                         gsems[b]).start()
                return 0
            lax.fori_loop(0, QB // NBUF - 1, rnd, 0)
            tds = []
            for b in range(NBUF):
                lj = QB - NBUF + b
                mk_g(base + lj, bufs[b], gsems[b]).wait()
                tds.append(pltpu.async_copy(
                    bufs[b], acc.at[dst_q.at[lj]], ssems[b], add=True))
            for d in tds:
                d.wait()
        plsc.subcore_barrier()
        pltpu.sync_copy(acc.at[pl.ds(s * rt, rt)],
                        out_hbm.at[pl.ds(chunk * N_P + s * rt, rt)])
        plsc.subcore_barrier()

    add_off((NCHUNK // NC) * c * N_P)
    chunk_pass((NCHUNK // NC) * c)
    for j in range(1, NCHUNK // NC):
        add_off(N_P)
        chunk_pass((NCHUNK // NC) * c + j)


_sc2 = functools.partial(
    pl.kernel,
    out_type=jax.ShapeDtypeStruct((NCHUNK * N_P, CW), jnp.float32),
    mesh=_mesh,
    scratch_types=(
        [pltpu.VMEM((EG,), jnp.int32),
         pltpu.VMEM((QB, 128), jnp.int32)]
        + [pltpu.VMEM((128, CW), jnp.float32) for _ in range(NBUF)]
        + [pltpu.VMEM_SHARED((N_P, CW), jnp.float32)]
        + [pltpu.SemaphoreType.DMA for _ in range(2 * NBUF + 1)]
    ),
    compiler_params=pltpu.CompilerParams(needs_layout_passes=False,
                                         use_tc_tiling_on_sc=False),
)(_sc2_body)


def _layer1_body(x_ref, aggx_ref, degp_ref, w1l_ref, b1_ref, w1r_ref,
                 h1c_ref, deg_ref):
    aggx = jnp.sum(aggx_ref[...], axis=0)
    deg = jnp.maximum(jnp.sum(degp_ref[...], axis=0), 1.0)
    a = aggx / deg
    h = (a[:, None] * w1l_ref[0] + b1_ref[0]
         + x_ref[...] * w1r_ref[0])
    h1c_ref[...] = jnp.maximum(h, 0.0)
    deg_ref[...] = deg[:, None]


def _dense_body(*refs):
    a_refs = refs[0:NCHUNK]
    h_refs = refs[NCHUNK:2 * NCHUNK]
    (deg_ref, wcl_ref, bc_ref, wcr_ref, w1_ref, bl1_ref, w2_ref, bl2_ref,
     w3_ref, bl3_ref, wf_ref, bf_ref, out_ref) = refs[2 * NCHUNK:]
    deg = jnp.maximum(deg_ref[...], 1.0)
    agg2 = jnp.concatenate([r[...] for r in a_refs], axis=1) / deg
    h1 = jnp.concatenate([r[...] for r in h_refs], axis=1)
    t = jnp.dot(agg2, wcl_ref[...], preferred_element_type=jnp.float32)
    t += jnp.dot(h1, wcr_ref[...], preferred_element_type=jnp.float32)
    t = jnp.maximum(t + bc_ref[...][None, :], 0.0)
    t = jnp.maximum(jnp.dot(t, w1_ref[...], preferred_element_type=jnp.float32)
                    + bl1_ref[...][None, :], 0.0)
    t = jnp.maximum(jnp.dot(t, w2_ref[...], preferred_element_type=jnp.float32)
                    + bl2_ref[...][None, :], 0.0)
    t = jnp.maximum(jnp.dot(t, w3_ref[...], preferred_element_type=jnp.float32)
                    + bl3_ref[...][None, :], 0.0)
    logits = jnp.dot(t, wf_ref[...], preferred_element_type=jnp.float32) \
        + bf_ref[...][None, :]
    m = jnp.max(logits, axis=1, keepdims=True)
    e = jnp.exp(logits - m)
    out_ref[...] = e / jnp.sum(e, axis=1, keepdims=True)


def kernel(x, edge_index, batch, W1l, b1, W1r, Wcl, bc, Wcr,
           Wlin1, blin1, Wlin2, blin2, Wlin3, blin3, Wfin, bfin):
    src = edge_index[0].astype(jnp.int32).reshape(NS, N)
    dst = edge_index[1].astype(jnp.int32).reshape(NS, N)
    src2 = jnp.pad(src, ((0, 0), (0, EG - N)))
    dst2 = jnp.pad(dst, ((0, 0), (0, EG - N)), constant_values=N)
    dst3 = dst2.reshape(NS, NB, 128)
    x_p = jnp.pad(x[:, 0], (0, N_P - N))

    aggx_p, deg_p = _sc1(x_p, src2, dst2)

    h1c, deg = pl.pallas_call(
        _layer1_body,
        grid=(NBLK, NCHUNK),
        in_specs=[
            pl.BlockSpec((BLK, 1), lambda i, c: (i, 0)),
            pl.BlockSpec((NC * NS, BLK), lambda i, c: (0, i)),
            pl.BlockSpec((NC * NS, BLK), lambda i, c: (0, i)),
            pl.BlockSpec((1, 1, CW), lambda i, c: (c, 0, 0)),
            pl.BlockSpec((1, 1, CW), lambda i, c: (c, 0, 0)),
            pl.BlockSpec((1, 1, CW), lambda i, c: (c, 0, 0)),
        ],
        out_specs=[
            pl.BlockSpec((BLK, CW), lambda i, c: (c * NBLK + i, 0)),
            pl.BlockSpec((BLK, 1), lambda i, c: (i, 0)),
        ],
        out_shape=[
            jax.ShapeDtypeStruct((NCHUNK * N_P, CW), jnp.float32),
            jax.ShapeDtypeStruct((N_P, 1), jnp.float32),
        ],
    )(jnp.pad(x, ((0, N_P - N), (0, 0))), aggx_p, deg_p,
      W1l.reshape(NCHUNK, 1, CW), b1.reshape(NCHUNK, 1, CW),
      W1r.reshape(NCHUNK, 1, CW))

    agg2 = _sc2(h1c, src2, dst3)

    def _rows(c):
        return pl.BlockSpec((BLK, CW), lambda i, c=c: (c * NBLK + i, 0))

    out = pl.pallas_call(
        _dense_body,
        grid=(NBLK,),
        in_specs=(
            [_rows(c) for c in range(NCHUNK)]
            + [_rows(c) for c in range(NCHUNK)]
            + [
                pl.BlockSpec((BLK, 1), lambda i: (i, 0)),
                pl.BlockSpec((L, L), lambda i: (0, 0)),
                pl.BlockSpec((L,), lambda i: (0,)),
                pl.BlockSpec((L, L), lambda i: (0, 0)),
                pl.BlockSpec((L, 256), lambda i: (0, 0)),
                pl.BlockSpec((256,), lambda i: (0,)),
                pl.BlockSpec((256, 128), lambda i: (0, 0)),
                pl.BlockSpec((128,), lambda i: (0,)),
                pl.BlockSpec((128, 64), lambda i: (0, 0)),
                pl.BlockSpec((64,), lambda i: (0,)),
                pl.BlockSpec((64, 2), lambda i: (0, 0)),
                pl.BlockSpec((2,), lambda i: (0,)),
            ]
        ),
        out_specs=pl.BlockSpec((BLK, 2), lambda i: (i, 0)),
        out_shape=jax.ShapeDtypeStruct((N_P, 2), jnp.float32),
    )(*([agg2] * NCHUNK), *([h1c] * NCHUNK), deg,
      Wcl, bc, Wcr, Wlin1, blin1, Wlin2, blin2, Wlin3, blin3, Wfin, bfin)
    return out[:N]
